# Initial kernel scaffold; baseline (speedup 1.0000x reference)
import jax
import jax.numpy as jnp
from jax.experimental import pallas as pl


def kernel(x, edge_index, W1, att_src1, att_dst1, b1, W2, att_src2, att_dst2, b2):
    raise NotImplementedError("write your pallas kernel here")


# speedup vs baseline: 1.6813x; 1.6813x over previous
import jax
import jax.numpy as jnp
from jax.experimental import pallas as pl


def _mm_kernel(x_ref, w_ref, h_ref):
    h_ref[...] = jnp.dot(x_ref[...], w_ref[...], preferred_element_type=jnp.float32)


def _matmul(x, W):
    N, D = x.shape
    BN = 1000
    return pl.pallas_call(
        _mm_kernel,
        grid=(N // BN,),
        in_specs=[pl.BlockSpec((BN, D), lambda i: (i, 0)),
                  pl.BlockSpec((D, W.shape[1]), lambda i: (0, 0))],
        out_specs=pl.BlockSpec((BN, W.shape[1]), lambda i: (i, 0)),
        out_shape=jax.ShapeDtypeStruct((N, W.shape[1]), jnp.float32),
    )(x, W)


def _gat_layer(x, src, dst, W, att_src, att_dst, b):
    n = x.shape[0]
    h = _matmul(x, W)
    a_src = h @ att_src
    a_dst = h @ att_dst
    c = jax.nn.leaky_relu(jnp.max(a_src) + jnp.max(a_dst), negative_slope=0.2)
    e = a_src[src] + a_dst[dst]
    e = jax.nn.leaky_relu(e, negative_slope=0.2)
    w = jnp.exp(e - c)
    denom = jax.ops.segment_sum(w, dst, num_segments=n)
    msg = h[src] * w[:, None]
    out = jax.ops.segment_sum(msg, dst, num_segments=n)
    return out / (denom[:, None] + 1e-16) + b


def kernel(x, edge_index, W1, att_src1, att_dst1, b1, W2, att_src2, att_dst2, b2):
    src = edge_index[0].astype(jnp.int32)
    dst = edge_index[1].astype(jnp.int32)
    h = _gat_layer(x, src, dst, W1, att_src1, att_dst1, b1)
    h = jax.nn.elu(h)
    return _gat_layer(h, src, dst, W2, att_src2, att_dst2, b2)


# speedup vs baseline: 25.3515x; 15.0783x over previous
import functools

import jax
import jax.numpy as jnp
from jax import lax
from jax.experimental import pallas as pl
from jax.experimental.pallas import tpu as pltpu
from jax.experimental.pallas import tpu_sc as plsc

N = 10000
D = 128
E = 320000
NC, NS, L = 2, 16, 16
NW = NC * NS
EPW = E // NW
CK = 80
NCHUNK = EPW // CK
RPT = 640
f32 = jnp.float32
i32 = jnp.int32


def _front_body(x_ref, w_ref, att_ref, h_ref, ap_ref, mx_ref):
    i = pl.program_id(0)
    h = jnp.dot(x_ref[...], w_ref[...], preferred_element_type=f32)
    h_ref[...] = h
    ap = jnp.dot(h, att_ref[...], preferred_element_type=f32)
    ap_ref[...] = ap
    cur = jnp.max(ap, axis=0, keepdims=True)

    @pl.when(i == 0)
    def _():
        mx_ref[...] = cur

    @pl.when(i != 0)
    def _():
        mx_ref[...] = jnp.maximum(mx_ref[...], cur)


def _front(x, W, att2):
    BN = 1000
    return pl.pallas_call(
        _front_body,
        grid=(N // BN,),
        in_specs=[pl.BlockSpec((BN, D), lambda i: (i, 0)),
                  pl.BlockSpec((D, D), lambda i: (0, 0)),
                  pl.BlockSpec((D, 16), lambda i: (0, 0))],
        out_specs=[pl.BlockSpec((BN, D), lambda i: (i, 0)),
                   pl.BlockSpec((BN, 16), lambda i: (i, 0)),
                   pl.BlockSpec((1, 16), lambda i: (0, 0))],
        out_shape=[jax.ShapeDtypeStruct((N, D), f32),
                   jax.ShapeDtypeStruct((N, 16), f32),
                   jax.ShapeDtypeStruct((1, 16), f32)],
    )(x, W, att2)


_sc_mesh = plsc.VectorSubcoreMesh(core_axis_name="c", subcore_axis_name="s",
                                  num_cores=NC, num_subcores=NS)


@functools.partial(
    pl.kernel,
    out_type=[jax.ShapeDtypeStruct((NC, N, D), f32),
              jax.ShapeDtypeStruct((NC, N, 16), f32)],
    mesh=_sc_mesh,
    compiler_params=pltpu.CompilerParams(needs_layout_passes=False,
                                         use_tc_tiling_on_sc=False),
    scratch_types=[
        pltpu.VMEM((16,), f32),
        pltpu.VMEM((2, CK), i32),
        pltpu.VMEM((CK, 16), f32),
        pltpu.VMEM((CK, 16), f32),
        pltpu.VMEM((CK, D), f32),
        pltpu.VMEM((CK, 16), f32),
        pltpu.VMEM_SHARED((N, D), f32),
        pltpu.VMEM_SHARED((N, 16), f32),
        pltpu.SemaphoreType.DMA,
        pltpu.SemaphoreType.DMA,
        pltpu.SemaphoreType.DMA,
    ],
)
def _sc_edge(ap_h, c_h, ei_h, h_h,
             out_h, den_h,
             c_t, sd_t, asg_t, adg_t, rows_t, wrep_t,
             out_acc, den_acc, sem, sem2, sem3):
    cid = lax.axis_index("c")
    sid = lax.axis_index("s")
    wid = cid * NS + sid
    zero16 = jnp.zeros((L,), f32)

    for k in range(CK):
        wrep_t[k] = zero16
        for q in range(D // L):
            rows_t[k, pl.ds(q * L, L)] = zero16
    nbase = sid * RPT
    for t in range(RPT // CK):
        rb = nbase + t * CK

        @pl.when(rb < N)
        def _():
            pltpu.sync_copy(rows_t, out_acc.at[pl.ds(rb, CK)])
            pltpu.sync_copy(wrep_t, den_acc.at[pl.ds(rb, CK)])

    pltpu.sync_copy(c_h, c_t)
    c_v = c_t[...]
    iot = lax.iota(i32, L)
    zeros_i = jnp.zeros((L,), i32)
    ones_i = jnp.ones((L,), i32)

    plsc.subcore_barrier()

    gdims = lax.GatherDimensionNumbers(
        offset_dims=(), collapsed_slice_dims=(0,), start_index_map=(0,))

    def _splat(v, k):
        idx = jnp.full((L, 1), k, i32)
        return lax.gather(v, idx, gdims, (1,),
                          mode=lax.GatherScatterMode.PROMISE_IN_BOUNDS)

    def chunk(j, carry):
        pltpu.sync_copy(ei_h.at[wid, j], sd_t)
        gcp = pltpu.async_copy(h_h.at[sd_t.at[0]], rows_t, sem)
        gcp2 = pltpu.async_copy(ap_h.at[sd_t.at[0]], asg_t, sem2)
        gcp3 = pltpu.async_copy(ap_h.at[sd_t.at[1]], adg_t, sem3)
        gcp2.wait()
        gcp3.wait()
        wvs = []
        for g in range(CK // L):
            kvec = g * L + iot
            e = (plsc.load_gather(asg_t, [kvec, zeros_i])
                 + plsc.load_gather(adg_t, [kvec, ones_i]))
            e = jnp.where(e >= 0.0, e, 0.2 * e) - c_v
            wvs.append(jnp.exp(e))
        gcp.wait()
        for g in range(CK // L):
            for k in range(L):
                wk = _splat(wvs[g], k)
                wrep_t[g * L + k] = wk
                r = g * L + k
                for q in range(D // L):
                    rows_t[r, pl.ds(q * L, L)] = rows_t[r, pl.ds(q * L, L)] * wk
        pltpu.sync_copy(rows_t, out_acc.at[sd_t.at[1]], add=True)
        pltpu.sync_copy(wrep_t, den_acc.at[sd_t.at[1]], add=True)
        return carry

    lax.fori_loop(0, NCHUNK, chunk, 0)

    plsc.subcore_barrier()

    for t in range(RPT // CK):
        rb = nbase + t * CK

        @pl.when(rb < N)
        def _():
            pltpu.sync_copy(out_acc.at[pl.ds(rb, CK)], rows_t)
            pltpu.sync_copy(rows_t, out_h.at[cid, pl.ds(rb, CK)])
            pltpu.sync_copy(den_acc.at[pl.ds(rb, CK)], wrep_t)
            pltpu.sync_copy(wrep_t, den_h.at[cid, pl.ds(rb, CK)])


def _combine_body(p_ref, d_ref, b_ref, o_ref, *, elu):
    s = p_ref[0] + p_ref[1]
    den = d_ref[0, :, 0:1] + d_ref[1, :, 0:1]
    y = s / (den + 1e-16) + b_ref[...]
    if elu:
        y = jnp.where(y > 0.0, y, jnp.exp(jnp.minimum(y, 0.0)) - 1.0)
    o_ref[...] = y


def _combine(p, d, b, elu):
    BN = 1000
    return pl.pallas_call(
        functools.partial(_combine_body, elu=elu),
        grid=(N // BN,),
        in_specs=[pl.BlockSpec((2, BN, D), lambda i: (0, i, 0)),
                  pl.BlockSpec((2, BN, 16), lambda i: (0, i, 0)),
                  pl.BlockSpec((1, D), lambda i: (0, 0))],
        out_specs=pl.BlockSpec((BN, D), lambda i: (i, 0)),
        out_shape=jax.ShapeDtypeStruct((N, D), f32),
    )(p, d, b)


def _layer(x, ei, W, att_s, att_d, b, elu):
    att2 = jnp.zeros((D, 16), f32).at[:, 0].set(att_s).at[:, 1].set(att_d)
    h, ap, mx = _front(x, W, att2)
    c = mx[0, 0] + mx[0, 1]
    c = jnp.where(c >= 0.0, c, 0.2 * c)
    c16 = jnp.broadcast_to(c, (16,))
    outp, denp = _sc_edge(ap, c16, ei, h)
    return _combine(outp, denp, b.reshape(1, D), elu)


def kernel(x, edge_index, W1, att_src1, att_dst1, b1, W2, att_src2, att_dst2, b2):
    src = edge_index[0].astype(i32).reshape(NW, NCHUNK, 1, CK)
    dst = edge_index[1].astype(i32).reshape(NW, NCHUNK, 1, CK)
    ei = jnp.concatenate([src, dst], axis=2)
    y = _layer(x, ei, W1, att_src1, att_dst1, b1, elu=True)
    return _layer(y, ei, W2, att_src2, att_dst2, b2, elu=False)


# speedup vs baseline: 29.8674x; 1.1781x over previous
import functools

import jax
import jax.numpy as jnp
from jax import lax
from jax.experimental import pallas as pl
from jax.experimental.pallas import tpu as pltpu
from jax.experimental.pallas import tpu_sc as plsc

N = 10000
D = 128
E = 320000
NC, NS, L = 2, 16, 16
NW = NC * NS
EPW = E // NW
CK = 80
NCHUNK = EPW // CK
RPT = 640
f32 = jnp.float32
i32 = jnp.int32


def _front_body(x_ref, w_ref, att_ref, h_ref, ap_ref, mx_ref):
    i = pl.program_id(0)
    h = jnp.dot(x_ref[...], w_ref[...], preferred_element_type=f32)
    h_ref[...] = h
    ap = jnp.dot(h, att_ref[...], preferred_element_type=f32)
    ap_ref[...] = ap
    cur = jnp.max(ap, axis=0, keepdims=True)

    @pl.when(i == 0)
    def _():
        mx_ref[...] = cur

    @pl.when(i != 0)
    def _():
        mx_ref[...] = jnp.maximum(mx_ref[...], cur)


def _front(x, W, att2):
    BN = 1000
    return pl.pallas_call(
        _front_body,
        grid=(N // BN,),
        in_specs=[pl.BlockSpec((BN, D), lambda i: (i, 0)),
                  pl.BlockSpec((D, D), lambda i: (0, 0)),
                  pl.BlockSpec((D, 16), lambda i: (0, 0))],
        out_specs=[pl.BlockSpec((BN, D), lambda i: (i, 0)),
                   pl.BlockSpec((BN, 16), lambda i: (i, 0)),
                   pl.BlockSpec((1, 16), lambda i: (0, 0))],
        out_shape=[jax.ShapeDtypeStruct((N, D), f32),
                   jax.ShapeDtypeStruct((N, 16), f32),
                   jax.ShapeDtypeStruct((1, 16), f32)],
    )(x, W, att2)


_sc_mesh = plsc.VectorSubcoreMesh(core_axis_name="c", subcore_axis_name="s",
                                  num_cores=NC, num_subcores=NS)


@functools.partial(
    pl.kernel,
    out_type=[jax.ShapeDtypeStruct((NC, N, D), f32),
              jax.ShapeDtypeStruct((NC, N, 16), f32)],
    mesh=_sc_mesh,
    compiler_params=pltpu.CompilerParams(needs_layout_passes=False,
                                         use_tc_tiling_on_sc=False),
    scratch_types=[
        pltpu.VMEM((16,), f32),
        pltpu.VMEM((2, CK), i32),
        pltpu.VMEM((2, CK), i32),
        pltpu.VMEM((CK, 16), f32),
        pltpu.VMEM((CK, 16), f32),
        pltpu.VMEM((CK, 16), f32),
        pltpu.VMEM((CK, 16), f32),
        pltpu.VMEM((CK, D), f32),
        pltpu.VMEM((CK, D), f32),
        pltpu.VMEM((CK, 16), f32),
        pltpu.SemaphoreType.DMA,
        pltpu.SemaphoreType.DMA,
        pltpu.SemaphoreType.DMA,
        pltpu.SemaphoreType.DMA,
        pltpu.VMEM_SHARED((N, D), f32),
        pltpu.VMEM_SHARED((N, 16), f32),
    ],
)
def _sc_edge(ap_h, c_h, ei_h, h_h,
             out_h, den_h,
             c_t, sd0, sd1, asg0, asg1, adg0, adg1, rows0, rows1, wrep_t,
             isem0, isem1, gsem0, gsem1,
             out_acc, den_acc):
    cid = lax.axis_index("c")
    sid = lax.axis_index("s")
    wid = cid * NS + sid
    zero16 = jnp.zeros((L,), f32)

    for k in range(CK):
        wrep_t[k] = zero16
        for q in range(D // L):
            rows0[k, pl.ds(q * L, L)] = zero16
    nbase = sid * RPT
    for t in range(RPT // CK):
        rb = nbase + t * CK

        @pl.when(rb < N)
        def _():
            pltpu.sync_copy(rows0, out_acc.at[pl.ds(rb, CK)])
            pltpu.sync_copy(wrep_t, den_acc.at[pl.ds(rb, CK)])

    pltpu.sync_copy(c_h, c_t)
    c_v = c_t[...]
    iot = lax.iota(i32, L)
    zeros_i = jnp.zeros((L,), i32)
    ones_i = jnp.ones((L,), i32)

    plsc.subcore_barrier()

    gdims = lax.GatherDimensionNumbers(
        offset_dims=(), collapsed_slice_dims=(0,), start_index_map=(0,))

    def _splat(v, k):
        idx = jnp.full((L, 1), k, i32)
        return lax.gather(v, idx, gdims, (1,),
                          mode=lax.GatherScatterMode.PROMISE_IN_BOUNDS)

    bufs = ((sd0, asg0, adg0, rows0, isem0, gsem0),
            (sd1, asg1, adg1, rows1, isem1, gsem1))

    def issue_gathers(b):
        sd, asg, adg, rows, _, gsem = b
        pltpu.async_copy(h_h.at[sd.at[0]], rows, gsem)
        pltpu.async_copy(ap_h.at[sd.at[0]], asg, gsem)
        pltpu.async_copy(ap_h.at[sd.at[1]], adg, gsem)

    def wait_gathers(b):
        sd, asg, adg, rows, _, gsem = b
        pltpu.make_async_copy(h_h.at[sd.at[0]], rows, gsem).wait()
        pltpu.make_async_copy(ap_h.at[sd.at[0]], asg, gsem).wait()
        pltpu.make_async_copy(ap_h.at[sd.at[1]], adg, gsem).wait()

    def process(j, b):
        sd, asg, adg, rows, _, _ = b
        wvs = []
        for g in range(CK // L):
            kvec = g * L + iot
            e = (plsc.load_gather(asg, [kvec, zeros_i])
                 + plsc.load_gather(adg, [kvec, ones_i]))
            e = jnp.where(e >= 0.0, e, 0.2 * e) - c_v
            wvs.append(jnp.exp(e))
        for g in range(CK // L):
            for k in range(L):
                r = g * L + k
                wk = _splat(wvs[g], k)
                wrep_t[r] = wk
                for q in range(D // L):
                    rows[r, pl.ds(q * L, L)] = rows[r, pl.ds(q * L, L)] * wk
        pltpu.sync_copy(rows, out_acc.at[sd.at[1]], add=True)
        pltpu.sync_copy(wrep_t, den_acc.at[sd.at[1]], add=True)

    def step(j, X, Y):
        sdY, isemY = Y[0], Y[4]
        pltpu.make_async_copy(ei_h.at[wid, j + 1], sdY, isemY).wait()
        issue_gathers(Y)
        wait_gathers(X)
        process(j, X)
        jp2 = jnp.minimum(j + 2, NCHUNK - 1)
        pltpu.async_copy(ei_h.at[wid, jp2], X[0], X[4])

    pltpu.sync_copy(ei_h.at[wid, 0], sd0)
    issue_gathers(bufs[0])
    pltpu.async_copy(ei_h.at[wid, 1], sd1, isem1)

    def pair(p, carry):
        j = 2 * p
        step(j, bufs[0], bufs[1])
        step(j + 1, bufs[1], bufs[0])
        return carry

    lax.fori_loop(0, (NCHUNK - 1) // 2, pair, 0)

    wait_gathers(bufs[0])
    process(NCHUNK - 1, bufs[0])
    pltpu.make_async_copy(ei_h.at[wid, NCHUNK - 1], sd1, isem1).wait()

    plsc.subcore_barrier()

    for t in range(RPT // CK):
        rb = nbase + t * CK

        @pl.when(rb < N)
        def _():
            pltpu.sync_copy(out_acc.at[pl.ds(rb, CK)], rows0)
            pltpu.sync_copy(rows0, out_h.at[cid, pl.ds(rb, CK)])
            pltpu.sync_copy(den_acc.at[pl.ds(rb, CK)], wrep_t)
            pltpu.sync_copy(wrep_t, den_h.at[cid, pl.ds(rb, CK)])


def _combine_body(p_ref, d_ref, b_ref, o_ref, *, elu):
    s = p_ref[0] + p_ref[1]
    den = d_ref[0, :, 0:1] + d_ref[1, :, 0:1]
    y = s / (den + 1e-16) + b_ref[...]
    if elu:
        y = jnp.where(y > 0.0, y, jnp.exp(jnp.minimum(y, 0.0)) - 1.0)
    o_ref[...] = y


def _combine(p, d, b, elu):
    BN = 1000
    return pl.pallas_call(
        functools.partial(_combine_body, elu=elu),
        grid=(N // BN,),
        in_specs=[pl.BlockSpec((2, BN, D), lambda i: (0, i, 0)),
                  pl.BlockSpec((2, BN, 16), lambda i: (0, i, 0)),
                  pl.BlockSpec((1, D), lambda i: (0, 0))],
        out_specs=pl.BlockSpec((BN, D), lambda i: (i, 0)),
        out_shape=jax.ShapeDtypeStruct((N, D), f32),
    )(p, d, b)


def _layer(x, ei, W, att_s, att_d, b, elu):
    att2 = jnp.zeros((D, 16), f32).at[:, 0].set(att_s).at[:, 1].set(att_d)
    h, ap, mx = _front(x, W, att2)
    c = mx[0, 0] + mx[0, 1]
    c = jnp.where(c >= 0.0, c, 0.2 * c)
    c16 = jnp.broadcast_to(c, (16,))
    outp, denp = _sc_edge(ap, c16, ei, h)
    return _combine(outp, denp, b.reshape(1, D), elu)


def kernel(x, edge_index, W1, att_src1, att_dst1, b1, W2, att_src2, att_dst2, b2):
    src = edge_index[0].astype(i32).reshape(NW, NCHUNK, 1, CK)
    dst = edge_index[1].astype(i32).reshape(NW, NCHUNK, 1, CK)
    ei = jnp.concatenate([src, dst], axis=2)
    y = _layer(x, ei, W1, att_src1, att_dst1, b1, elu=True)
    return _layer(y, ei, W2, att_src2, att_dst2, b2, elu=False)


# speedup vs baseline: 35.1661x; 1.1774x over previous
import functools

import jax
import jax.numpy as jnp
from jax import lax
from jax.experimental import pallas as pl
from jax.experimental.pallas import tpu as pltpu
from jax.experimental.pallas import tpu_sc as plsc

N = 10000
D = 128
E = 320000
NC, NS, L = 2, 16, 16
NW = NC * NS
EPW = E // NW
CK = 80
NCHUNK = EPW // CK
RPT = 640
f32 = jnp.float32
i32 = jnp.int32


def _front_body(x_ref, w_ref, att_ref, h_ref, ap_ref, mx_ref):
    i = pl.program_id(0)
    h = jnp.dot(x_ref[...], w_ref[...], preferred_element_type=f32)
    h_ref[...] = h
    ap = jnp.dot(h, att_ref[...], preferred_element_type=f32)
    ap_ref[...] = ap
    cur = jnp.max(ap, axis=0, keepdims=True)

    @pl.when(i == 0)
    def _():
        mx_ref[...] = cur

    @pl.when(i != 0)
    def _():
        mx_ref[...] = jnp.maximum(mx_ref[...], cur)


def _front(x, W, att2):
    BN = 1000
    return pl.pallas_call(
        _front_body,
        grid=(N // BN,),
        in_specs=[pl.BlockSpec((BN, D), lambda i: (i, 0)),
                  pl.BlockSpec((D, D), lambda i: (0, 0)),
                  pl.BlockSpec((D, 16), lambda i: (0, 0))],
        out_specs=[pl.BlockSpec((BN, D), lambda i: (i, 0)),
                   pl.BlockSpec((BN, 16), lambda i: (i, 0)),
                   pl.BlockSpec((1, 16), lambda i: (0, 0))],
        out_shape=[jax.ShapeDtypeStruct((N, D), f32),
                   jax.ShapeDtypeStruct((N, 16), f32),
                   jax.ShapeDtypeStruct((1, 16), f32)],
    )(x, W, att2)


_sc_mesh = plsc.VectorSubcoreMesh(core_axis_name="c", subcore_axis_name="s",
                                  num_cores=NC, num_subcores=NS)


@functools.partial(
    pl.kernel,
    out_type=[jax.ShapeDtypeStruct((NC, N, D), f32),
              jax.ShapeDtypeStruct((NC, N, 16), f32)],
    mesh=_sc_mesh,
    compiler_params=pltpu.CompilerParams(needs_layout_passes=False,
                                         use_tc_tiling_on_sc=False),
    scratch_types=[
        pltpu.VMEM((16,), f32),
        pltpu.VMEM((2, CK), i32),
        pltpu.VMEM((2, CK), i32),
        pltpu.VMEM((CK, 16), f32),
        pltpu.VMEM((CK, 16), f32),
        pltpu.VMEM((CK, 16), f32),
        pltpu.VMEM((CK, 16), f32),
        pltpu.VMEM((CK, D), f32),
        pltpu.VMEM((CK, D), f32),
        pltpu.VMEM((CK, 16), f32),
        pltpu.VMEM((CK, 16), f32),
        pltpu.VMEM((1, CK), i32),
        pltpu.VMEM((1, CK), i32),
        pltpu.SemaphoreType.DMA,
        pltpu.SemaphoreType.DMA,
        pltpu.SemaphoreType.DMA,
        pltpu.SemaphoreType.DMA,
        pltpu.SemaphoreType.DMA,
        pltpu.SemaphoreType.DMA,
        pltpu.VMEM_SHARED((N, D), f32),
        pltpu.VMEM_SHARED((N, 16), f32),
    ],
)
def _sc_edge(ap_h, c_h, ei_h, h_h,
             out_h, den_h,
             c_t, sd0, sd1, asg0, asg1, adg0, adg1, rows0, rows1,
             wrep0, wrep1, scx0, scx1,
             isem0, isem1, gsem0, gsem1, ssem0, ssem1,
             out_acc, den_acc):
    cid = lax.axis_index("c")
    sid = lax.axis_index("s")
    wid = cid * NS + sid
    zero16 = jnp.zeros((L,), f32)

    for k in range(CK):
        wrep0[k] = zero16
        for q in range(D // L):
            rows0[k, pl.ds(q * L, L)] = zero16
    nbase = sid * RPT
    for t in range(RPT // CK):
        rb = nbase + t * CK

        @pl.when(rb < N)
        def _():
            pltpu.sync_copy(rows0, out_acc.at[pl.ds(rb, CK)])
            pltpu.sync_copy(wrep0, den_acc.at[pl.ds(rb, CK)])

    pltpu.sync_copy(c_h, c_t)
    c_v = c_t[...]
    iot = lax.iota(i32, L)
    zeros_i = jnp.zeros((L,), i32)
    ones_i = jnp.ones((L,), i32)

    plsc.subcore_barrier()

    gdims = lax.GatherDimensionNumbers(
        offset_dims=(), collapsed_slice_dims=(0,), start_index_map=(0,))

    def _splat(v, k):
        idx = jnp.full((L, 1), k, i32)
        return lax.gather(v, idx, gdims, (1,),
                          mode=lax.GatherScatterMode.PROMISE_IN_BOUNDS)

    bufs = ((sd0, asg0, adg0, rows0, wrep0, scx0, isem0, gsem0, ssem0),
            (sd1, asg1, adg1, rows1, wrep1, scx1, isem1, gsem1, ssem1))

    def issue_gathers(b):
        sd, asg, adg, rows = b[0], b[1], b[2], b[3]
        gsem = b[7]
        pltpu.async_copy(h_h.at[sd.at[0]], rows, gsem)
        pltpu.async_copy(ap_h.at[sd.at[0]], asg, gsem)
        pltpu.async_copy(ap_h.at[sd.at[1]], adg, gsem)

    def wait_gathers(b):
        sd, asg, adg, rows = b[0], b[1], b[2], b[3]
        gsem = b[7]
        pltpu.make_async_copy(h_h.at[sd.at[0]], rows, gsem).wait()
        pltpu.make_async_copy(ap_h.at[sd.at[0]], asg, gsem).wait()
        pltpu.make_async_copy(ap_h.at[sd.at[1]], adg, gsem).wait()

    def issue_scatters(b):
        rows, wrep, scx, ssem = b[3], b[4], b[5], b[8]
        pltpu.async_copy(rows, out_acc.at[scx.at[0]], ssem, add=True)
        pltpu.async_copy(wrep, den_acc.at[scx.at[0]], ssem, add=True)

    def wait_scatters(b):
        rows, wrep, scx, ssem = b[3], b[4], b[5], b[8]
        pltpu.make_async_copy(rows, out_acc.at[scx.at[0]], ssem).wait()
        pltpu.make_async_copy(wrep, den_acc.at[scx.at[0]], ssem).wait()

    def process(b):
        sd, asg, adg, rows, wrep, scx = b[0], b[1], b[2], b[3], b[4], b[5]
        wvs = []
        for g in range(CK // L):
            kvec = g * L + iot
            scx[0, pl.ds(g * L, L)] = sd[1, pl.ds(g * L, L)]
            e = (plsc.load_gather(asg, [kvec, zeros_i])
                 + plsc.load_gather(adg, [kvec, ones_i]))
            e = jnp.where(e >= 0.0, e, 0.2 * e) - c_v
            wvs.append(jnp.exp(e))
        for g in range(CK // L):
            for k in range(L):
                r = g * L + k
                wk = _splat(wvs[g], k)
                wrep[r] = wk
                for q in range(D // L):
                    rows[r, pl.ds(q * L, L)] = rows[r, pl.ds(q * L, L)] * wk
        issue_scatters(b)

    def step(j, X, Y, wait_scatter):
        if wait_scatter:
            wait_scatters(Y)
        pltpu.make_async_copy(ei_h.at[wid, j + 1], Y[0], Y[6]).wait()
        issue_gathers(Y)
        wait_gathers(X)
        process(X)
        jp2 = jnp.minimum(j + 2, NCHUNK - 1)
        pltpu.async_copy(ei_h.at[wid, jp2], X[0], X[6])

    pltpu.sync_copy(ei_h.at[wid, 0], sd0)
    issue_gathers(bufs[0])
    pltpu.async_copy(ei_h.at[wid, 1], sd1, isem1)
    step(0, bufs[0], bufs[1], wait_scatter=False)
    step(1, bufs[1], bufs[0], wait_scatter=True)

    def pair(p, carry):
        j = 2 * p
        step(j, bufs[0], bufs[1], wait_scatter=True)
        step(j + 1, bufs[1], bufs[0], wait_scatter=True)
        return carry

    lax.fori_loop(1, (NCHUNK - 1) // 2, pair, 0)

    wait_scatters(bufs[1])
    wait_gathers(bufs[0])
    process(bufs[0])
    wait_scatters(bufs[0])
    pltpu.make_async_copy(ei_h.at[wid, NCHUNK - 1], sd1, isem1).wait()

    plsc.subcore_barrier()

    for t in range(RPT // CK):
        rb = nbase + t * CK

        @pl.when(rb < N)
        def _():
            pltpu.sync_copy(out_acc.at[pl.ds(rb, CK)], rows0)
            pltpu.sync_copy(rows0, out_h.at[cid, pl.ds(rb, CK)])
            pltpu.sync_copy(den_acc.at[pl.ds(rb, CK)], wrep0)
            pltpu.sync_copy(wrep0, den_h.at[cid, pl.ds(rb, CK)])


def _combine_body(p_ref, d_ref, b_ref, o_ref, *, elu):
    s = p_ref[0] + p_ref[1]
    den = d_ref[0, :, 0:1] + d_ref[1, :, 0:1]
    y = s / (den + 1e-16) + b_ref[...]
    if elu:
        y = jnp.where(y > 0.0, y, jnp.exp(jnp.minimum(y, 0.0)) - 1.0)
    o_ref[...] = y


def _combine(p, d, b, elu):
    BN = 1000
    return pl.pallas_call(
        functools.partial(_combine_body, elu=elu),
        grid=(N // BN,),
        in_specs=[pl.BlockSpec((2, BN, D), lambda i: (0, i, 0)),
                  pl.BlockSpec((2, BN, 16), lambda i: (0, i, 0)),
                  pl.BlockSpec((1, D), lambda i: (0, 0))],
        out_specs=pl.BlockSpec((BN, D), lambda i: (i, 0)),
        out_shape=jax.ShapeDtypeStruct((N, D), f32),
    )(p, d, b)


def _layer(x, ei, W, att_s, att_d, b, elu):
    att2 = jnp.zeros((D, 16), f32).at[:, 0].set(att_s).at[:, 1].set(att_d)
    h, ap, mx = _front(x, W, att2)
    c = mx[0, 0] + mx[0, 1]
    c = jnp.where(c >= 0.0, c, 0.2 * c)
    c16 = jnp.broadcast_to(c, (16,))
    outp, denp = _sc_edge(ap, c16, ei, h)
    return _combine(outp, denp, b.reshape(1, D), elu)


def kernel(x, edge_index, W1, att_src1, att_dst1, b1, W2, att_src2, att_dst2, b2):
    src = edge_index[0].astype(i32).reshape(NW, NCHUNK, 1, CK)
    dst = edge_index[1].astype(i32).reshape(NW, NCHUNK, 1, CK)
    ei = jnp.concatenate([src, dst], axis=2)
    y = _layer(x, ei, W1, att_src1, att_dst1, b1, elu=True)
    return _layer(y, ei, W2, att_src2, att_dst2, b2, elu=False)


# speedup vs baseline: 35.5475x; 1.0108x over previous
import functools

import jax
import jax.numpy as jnp
from jax import lax
from jax.experimental import pallas as pl
from jax.experimental.pallas import tpu as pltpu
from jax.experimental.pallas import tpu_sc as plsc

N = 10000
D = 128
E = 320000
NC, NS, L = 2, 16, 16
NW = NC * NS
EPW = E // NW
CK = 80
NCHUNK = EPW // CK
RPT = 640
f32 = jnp.float32
i32 = jnp.int32


def _front_body(x_ref, w_ref, att_ref, h_ref, ap_ref, mx_ref):
    i = pl.program_id(0)
    h = jnp.dot(x_ref[...], w_ref[...], preferred_element_type=f32)
    h_ref[...] = h
    ap = jnp.dot(h, att_ref[...], preferred_element_type=f32)
    ap_ref[...] = ap
    cur = jnp.max(ap, axis=0, keepdims=True)

    @pl.when(i == 0)
    def _():
        mx_ref[...] = cur

    @pl.when(i != 0)
    def _():
        mx_ref[...] = jnp.maximum(mx_ref[...], cur)


def _front(x, W, att2):
    BN = 1000
    return pl.pallas_call(
        _front_body,
        grid=(N // BN,),
        in_specs=[pl.BlockSpec((BN, D), lambda i: (i, 0)),
                  pl.BlockSpec((D, D), lambda i: (0, 0)),
                  pl.BlockSpec((D, 16), lambda i: (0, 0))],
        out_specs=[pl.BlockSpec((BN, D), lambda i: (i, 0)),
                   pl.BlockSpec((BN, 16), lambda i: (i, 0)),
                   pl.BlockSpec((1, 16), lambda i: (0, 0))],
        out_shape=[jax.ShapeDtypeStruct((N, D), f32),
                   jax.ShapeDtypeStruct((N, 16), f32),
                   jax.ShapeDtypeStruct((1, 16), f32)],
    )(x, W, att2)


_sc_mesh = plsc.VectorSubcoreMesh(core_axis_name="c", subcore_axis_name="s",
                                  num_cores=NC, num_subcores=NS)


@functools.partial(
    pl.kernel,
    out_type=[jax.ShapeDtypeStruct((NC, N, D), f32),
              jax.ShapeDtypeStruct((NC, N, 16), f32)],
    mesh=_sc_mesh,
    compiler_params=pltpu.CompilerParams(needs_layout_passes=False,
                                         use_tc_tiling_on_sc=False),
    scratch_types=[
        pltpu.VMEM((16,), f32),
        pltpu.VMEM((2, CK), i32),
        pltpu.VMEM((2, CK), i32),
        pltpu.VMEM((CK, 16), f32),
        pltpu.VMEM((CK, 16), f32),
        pltpu.VMEM((CK, 16), f32),
        pltpu.VMEM((CK, 16), f32),
        pltpu.VMEM((CK, D), f32),
        pltpu.VMEM((CK, D), f32),
        pltpu.VMEM((CK, 16), f32),
        pltpu.VMEM((CK, 16), f32),
        pltpu.VMEM((1, CK), i32),
        pltpu.VMEM((1, CK), i32),
        pltpu.SemaphoreType.DMA,
        pltpu.SemaphoreType.DMA,
        pltpu.SemaphoreType.DMA,
        pltpu.SemaphoreType.DMA,
        pltpu.SemaphoreType.DMA,
        pltpu.SemaphoreType.DMA,
        pltpu.VMEM_SHARED((N, D), f32),
        pltpu.VMEM_SHARED((N, 16), f32),
    ],
)
def _sc_edge(ap_h, c_h, ei_h, h_h,
             out_h, den_h,
             c_t, sd0, sd1, asg0, asg1, adg0, adg1, rows0, rows1,
             wrep0, wrep1, scx0, scx1,
             isem0, isem1, gsem0, gsem1, ssem0, ssem1,
             out_acc, den_acc):
    cid = lax.axis_index("c")
    sid = lax.axis_index("s")
    wid = cid * NS + sid
    zero16 = jnp.zeros((L,), f32)

    for k in range(CK):
        wrep0[k] = zero16
        for q in range(D // L):
            rows0[k, pl.ds(q * L, L)] = zero16
    nbase = sid * RPT
    for t in range(RPT // CK):
        rb = nbase + t * CK

        @pl.when(rb < N)
        def _():
            pltpu.sync_copy(rows0, out_acc.at[pl.ds(rb, CK)])
            pltpu.sync_copy(wrep0, den_acc.at[pl.ds(rb, CK)])

    pltpu.sync_copy(c_h, c_t)
    c_v = c_t[...]
    iot = lax.iota(i32, L)
    zeros_i = jnp.zeros((L,), i32)
    ones_i = jnp.ones((L,), i32)

    plsc.subcore_barrier()

    gdims = lax.GatherDimensionNumbers(
        offset_dims=(), collapsed_slice_dims=(0,), start_index_map=(0,))

    def _splat(v, k):
        idx = jnp.full((L, 1), k, i32)
        return lax.gather(v, idx, gdims, (1,),
                          mode=lax.GatherScatterMode.PROMISE_IN_BOUNDS)

    bufs = ((sd0, asg0, adg0, rows0, wrep0, scx0, isem0, gsem0, ssem0),
            (sd1, asg1, adg1, rows1, wrep1, scx1, isem1, gsem1, ssem1))

    def issue_gathers(b):
        sd, asg, adg, rows = b[0], b[1], b[2], b[3]
        gsem = b[7]
        pltpu.async_copy(h_h.at[sd.at[0]], rows, gsem)
        pltpu.async_copy(ap_h.at[sd.at[0]], asg, gsem)
        pltpu.async_copy(ap_h.at[sd.at[1]], adg, gsem)

    def wait_gathers(b):
        sd, asg, adg, rows = b[0], b[1], b[2], b[3]
        gsem = b[7]
        pltpu.make_async_copy(h_h.at[sd.at[0]], rows, gsem).wait()
        pltpu.make_async_copy(ap_h.at[sd.at[0]], asg, gsem).wait()
        pltpu.make_async_copy(ap_h.at[sd.at[1]], adg, gsem).wait()

    def issue_scatters(b):
        rows, wrep, scx, ssem = b[3], b[4], b[5], b[8]
        pltpu.async_copy(rows, out_acc.at[scx.at[0]], ssem, add=True)
        pltpu.async_copy(wrep, den_acc.at[scx.at[0]], ssem, add=True)

    def wait_scatters(b):
        rows, wrep, scx, ssem = b[3], b[4], b[5], b[8]
        pltpu.make_async_copy(rows, out_acc.at[scx.at[0]], ssem).wait()
        pltpu.make_async_copy(wrep, den_acc.at[scx.at[0]], ssem).wait()

    def process(b):
        sd, asg, adg, rows, wrep, scx = b[0], b[1], b[2], b[3], b[4], b[5]
        wvs = []
        for g in range(CK // L):
            kvec = g * L + iot
            scx[0, pl.ds(g * L, L)] = sd[1, pl.ds(g * L, L)]
            e = (plsc.load_gather(asg, [kvec, zeros_i])
                 + plsc.load_gather(adg, [kvec, ones_i]))
            e = jnp.where(e >= 0.0, e, 0.2 * e) - c_v
            wvs.append(jnp.exp(e))
        for g in range(CK // L):
            for k in range(L):
                r = g * L + k
                wk = _splat(wvs[g], k)
                wrep[r] = wk
                for q in range(D // L):
                    rows[r, pl.ds(q * L, L)] = rows[r, pl.ds(q * L, L)] * wk
        issue_scatters(b)

    def step(j, X, Y, wait_scatter):
        if wait_scatter:
            wait_scatters(Y)
        pltpu.make_async_copy(ei_h.at[wid, j + 1], Y[0], Y[6]).wait()
        issue_gathers(Y)
        wait_gathers(X)
        process(X)
        jp2 = jnp.minimum(j + 2, NCHUNK - 1)
        pltpu.async_copy(ei_h.at[wid, jp2], X[0], X[6])

    pltpu.sync_copy(ei_h.at[wid, 0], sd0)
    issue_gathers(bufs[0])
    pltpu.async_copy(ei_h.at[wid, 1], sd1, isem1)
    step(0, bufs[0], bufs[1], wait_scatter=False)
    step(1, bufs[1], bufs[0], wait_scatter=True)

    def pair(p, carry):
        j = 2 * p
        step(j, bufs[0], bufs[1], wait_scatter=True)
        step(j + 1, bufs[1], bufs[0], wait_scatter=True)
        return carry

    lax.fori_loop(1, (NCHUNK - 1) // 2, pair, 0)

    wait_scatters(bufs[1])
    wait_gathers(bufs[0])
    process(bufs[0])
    wait_scatters(bufs[0])
    pltpu.make_async_copy(ei_h.at[wid, NCHUNK - 1], sd1, isem1).wait()

    plsc.subcore_barrier()

    for t in range(RPT // CK):
        rb = nbase + t * CK

        @pl.when(rb < N)
        def _():
            pltpu.sync_copy(out_acc.at[pl.ds(rb, CK)], rows0)
            pltpu.sync_copy(rows0, out_h.at[cid, pl.ds(rb, CK)])
            pltpu.sync_copy(den_acc.at[pl.ds(rb, CK)], wrep0)
            pltpu.sync_copy(wrep0, den_h.at[cid, pl.ds(rb, CK)])


def _mid_body(p_ref, d_ref, b_ref, w_ref, att_ref, h_ref, ap_ref, mx_ref):
    i = pl.program_id(0)
    s = p_ref[0] + p_ref[1]
    den = d_ref[0, :, 0:1] + d_ref[1, :, 0:1]
    y = s / (den + 1e-16) + b_ref[...]
    y = jnp.where(y > 0.0, y, jnp.exp(jnp.minimum(y, 0.0)) - 1.0)
    h = jnp.dot(y, w_ref[...], preferred_element_type=f32)
    h_ref[...] = h
    ap = jnp.dot(h, att_ref[...], preferred_element_type=f32)
    ap_ref[...] = ap
    cur = jnp.max(ap, axis=0, keepdims=True)

    @pl.when(i == 0)
    def _():
        mx_ref[...] = cur

    @pl.when(i != 0)
    def _():
        mx_ref[...] = jnp.maximum(mx_ref[...], cur)


def _mid(p, d, b, W, att2):
    BN = 1000
    return pl.pallas_call(
        _mid_body,
        grid=(N // BN,),
        in_specs=[pl.BlockSpec((2, BN, D), lambda i: (0, i, 0)),
                  pl.BlockSpec((2, BN, 16), lambda i: (0, i, 0)),
                  pl.BlockSpec((1, D), lambda i: (0, 0)),
                  pl.BlockSpec((D, D), lambda i: (0, 0)),
                  pl.BlockSpec((D, 16), lambda i: (0, 0))],
        out_specs=[pl.BlockSpec((BN, D), lambda i: (i, 0)),
                   pl.BlockSpec((BN, 16), lambda i: (i, 0)),
                   pl.BlockSpec((1, 16), lambda i: (0, 0))],
        out_shape=[jax.ShapeDtypeStruct((N, D), f32),
                   jax.ShapeDtypeStruct((N, 16), f32),
                   jax.ShapeDtypeStruct((1, 16), f32)],
    )(p, d, b, W, att2)


def _combine_body(p_ref, d_ref, b_ref, o_ref, *, elu):
    s = p_ref[0] + p_ref[1]
    den = d_ref[0, :, 0:1] + d_ref[1, :, 0:1]
    y = s / (den + 1e-16) + b_ref[...]
    if elu:
        y = jnp.where(y > 0.0, y, jnp.exp(jnp.minimum(y, 0.0)) - 1.0)
    o_ref[...] = y


def _combine(p, d, b, elu):
    BN = 1000
    return pl.pallas_call(
        functools.partial(_combine_body, elu=elu),
        grid=(N // BN,),
        in_specs=[pl.BlockSpec((2, BN, D), lambda i: (0, i, 0)),
                  pl.BlockSpec((2, BN, 16), lambda i: (0, i, 0)),
                  pl.BlockSpec((1, D), lambda i: (0, 0))],
        out_specs=pl.BlockSpec((BN, D), lambda i: (i, 0)),
        out_shape=jax.ShapeDtypeStruct((N, D), f32),
    )(p, d, b)


def _shift(mx):
    c = mx[0, 0] + mx[0, 1]
    c = jnp.where(c >= 0.0, c, 0.2 * c)
    return jnp.broadcast_to(c, (16,))


def kernel(x, edge_index, W1, att_src1, att_dst1, b1, W2, att_src2, att_dst2, b2):
    src = edge_index[0].astype(i32).reshape(NW, NCHUNK, 1, CK)
    dst = edge_index[1].astype(i32).reshape(NW, NCHUNK, 1, CK)
    ei = jnp.concatenate([src, dst], axis=2)
    att21 = jnp.zeros((D, 16), f32).at[:, 0].set(att_src1).at[:, 1].set(att_dst1)
    att22 = jnp.zeros((D, 16), f32).at[:, 0].set(att_src2).at[:, 1].set(att_dst2)

    h1, ap1, mx1 = _front(x, W1, att21)
    outp1, denp1 = _sc_edge(ap1, _shift(mx1), ei, h1)
    h2, ap2, mx2 = _mid(outp1, denp1, b1.reshape(1, D), W2, att22)
    outp2, denp2 = _sc_edge(ap2, _shift(mx2), ei, h2)
    return _combine(outp2, denp2, b2.reshape(1, D), elu=False)


# speedup vs baseline: 36.0006x; 1.0127x over previous
import functools

import jax
import jax.numpy as jnp
from jax import lax
from jax.experimental import pallas as pl
from jax.experimental.pallas import tpu as pltpu
from jax.experimental.pallas import tpu_sc as plsc

N = 10000
D = 128
E = 320000
NC, NS, L = 2, 16, 16
NW = NC * NS
EPW = E // NW
CK = 80
NCHUNK = EPW // CK
RPT = 640
f32 = jnp.float32
i32 = jnp.int32


def _front_body(x_ref, w_ref, att_ref, h_ref, ap_ref, mx_ref):
    i = pl.program_id(0)
    h = jnp.dot(x_ref[...], w_ref[...], preferred_element_type=f32)
    h_ref[...] = h
    ap = jnp.dot(h, att_ref[...], preferred_element_type=f32)
    ap_ref[...] = ap
    cur = jnp.max(ap, axis=0, keepdims=True)

    @pl.when(i == 0)
    def _():
        mx_ref[...] = cur

    @pl.when(i != 0)
    def _():
        mx_ref[...] = jnp.maximum(mx_ref[...], cur)


def _front(x, W, att2):
    BN = 1000
    return pl.pallas_call(
        _front_body,
        grid=(N // BN,),
        in_specs=[pl.BlockSpec((BN, D), lambda i: (i, 0)),
                  pl.BlockSpec((D, D), lambda i: (0, 0)),
                  pl.BlockSpec((D, 16), lambda i: (0, 0))],
        out_specs=[pl.BlockSpec((BN, D), lambda i: (i, 0)),
                   pl.BlockSpec((BN, 16), lambda i: (i, 0)),
                   pl.BlockSpec((1, 16), lambda i: (0, 0))],
        out_shape=[jax.ShapeDtypeStruct((N, D), f32),
                   jax.ShapeDtypeStruct((N, 16), f32),
                   jax.ShapeDtypeStruct((1, 16), f32)],
    )(x, W, att2)


_sc_mesh = plsc.VectorSubcoreMesh(core_axis_name="c", subcore_axis_name="s",
                                  num_cores=NC, num_subcores=NS)


@functools.partial(
    pl.kernel,
    out_type=[jax.ShapeDtypeStruct((NC, N, D), f32),
              jax.ShapeDtypeStruct((NC, N, 16), f32)],
    mesh=_sc_mesh,
    compiler_params=pltpu.CompilerParams(needs_layout_passes=False,
                                         use_tc_tiling_on_sc=False),
    scratch_types=[
        pltpu.VMEM((16,), f32),
        pltpu.VMEM((1, 2 * CK), i32),
        pltpu.VMEM((1, 2 * CK), i32),
        pltpu.VMEM((2 * CK, 16), f32),
        pltpu.VMEM((2 * CK, 16), f32),
        pltpu.VMEM((CK, D), f32),
        pltpu.VMEM((CK, D), f32),
        pltpu.VMEM((CK, 16), f32),
        pltpu.VMEM((CK, 16), f32),
        pltpu.VMEM((1, CK), i32),
        pltpu.VMEM((1, CK), i32),
        pltpu.SemaphoreType.DMA,
        pltpu.SemaphoreType.DMA,
        pltpu.SemaphoreType.DMA,
        pltpu.SemaphoreType.DMA,
        pltpu.SemaphoreType.DMA,
        pltpu.SemaphoreType.DMA,
        pltpu.VMEM_SHARED((N, D), f32),
        pltpu.VMEM_SHARED((N, 16), f32),
    ],
)
def _sc_edge(ap_h, c_h, ei_h, h_h,
             out_h, den_h,
             c_t, sd0, sd1, asg0, asg1, rows0, rows1,
             wrep0, wrep1, scx0, scx1,
             isem0, isem1, gsem0, gsem1, ssem0, ssem1,
             out_acc, den_acc):
    cid = lax.axis_index("c")
    sid = lax.axis_index("s")
    wid = cid * NS + sid
    zero16 = jnp.zeros((L,), f32)

    for k in range(CK):
        wrep0[k] = zero16
        for q in range(D // L):
            rows0[k, pl.ds(q * L, L)] = zero16
    nbase = sid * RPT
    for t in range(RPT // CK):
        rb = nbase + t * CK

        @pl.when(rb < N)
        def _():
            pltpu.sync_copy(rows0, out_acc.at[pl.ds(rb, CK)])
            pltpu.sync_copy(wrep0, den_acc.at[pl.ds(rb, CK)])

    pltpu.sync_copy(c_h, c_t)
    c_v = c_t[...]
    iot = lax.iota(i32, L)
    zeros_i = jnp.zeros((L,), i32)
    ones_i = jnp.ones((L,), i32)

    plsc.subcore_barrier()

    gdims = lax.GatherDimensionNumbers(
        offset_dims=(), collapsed_slice_dims=(0,), start_index_map=(0,))

    def _splat(v, k):
        idx = jnp.full((L, 1), k, i32)
        return lax.gather(v, idx, gdims, (1,),
                          mode=lax.GatherScatterMode.PROMISE_IN_BOUNDS)

    bufs = ((sd0, asg0, None, rows0, wrep0, scx0, isem0, gsem0, ssem0),
            (sd1, asg1, None, rows1, wrep1, scx1, isem1, gsem1, ssem1))

    def issue_gathers(b):
        sd, asg, rows = b[0], b[1], b[3]
        gsem = b[7]
        pltpu.async_copy(h_h.at[sd.at[0, pl.ds(0, CK)]], rows, gsem)
        pltpu.async_copy(ap_h.at[sd.at[0]], asg, gsem)

    def wait_gathers(b):
        sd, asg, rows = b[0], b[1], b[3]
        gsem = b[7]
        pltpu.make_async_copy(h_h.at[sd.at[0, pl.ds(0, CK)]], rows, gsem).wait()
        pltpu.make_async_copy(ap_h.at[sd.at[0]], asg, gsem).wait()

    def issue_scatters(b):
        rows, wrep, scx, ssem = b[3], b[4], b[5], b[8]
        pltpu.async_copy(rows, out_acc.at[scx.at[0]], ssem, add=True)
        pltpu.async_copy(wrep, den_acc.at[scx.at[0]], ssem, add=True)

    def wait_scatters(b):
        rows, wrep, scx, ssem = b[3], b[4], b[5], b[8]
        pltpu.make_async_copy(rows, out_acc.at[scx.at[0]], ssem).wait()
        pltpu.make_async_copy(wrep, den_acc.at[scx.at[0]], ssem).wait()

    def process(b):
        sd, asg, rows, wrep, scx = b[0], b[1], b[3], b[4], b[5]
        wvs = []
        for g in range(CK // L):
            kvec = g * L + iot
            scx[0, pl.ds(g * L, L)] = sd[0, pl.ds(CK + g * L, L)]
            e = (plsc.load_gather(asg, [kvec, zeros_i])
                 + plsc.load_gather(asg, [CK + kvec, ones_i]))
            e = jnp.where(e >= 0.0, e, 0.2 * e) - c_v
            wvs.append(jnp.exp(e))
        for g in range(CK // L):
            for k in range(L):
                r = g * L + k
                wk = _splat(wvs[g], k)
                wrep[r] = wk
                for q in range(D // L):
                    rows[r, pl.ds(q * L, L)] = rows[r, pl.ds(q * L, L)] * wk
        issue_scatters(b)

    def step(j, X, Y, wait_scatter):
        if wait_scatter:
            wait_scatters(Y)
        pltpu.make_async_copy(ei_h.at[wid, j + 1], Y[0], Y[6]).wait()
        issue_gathers(Y)
        wait_gathers(X)
        process(X)
        jp2 = jnp.minimum(j + 2, NCHUNK - 1)
        pltpu.async_copy(ei_h.at[wid, jp2], X[0], X[6])

    pltpu.sync_copy(ei_h.at[wid, 0], sd0)
    issue_gathers(bufs[0])
    pltpu.async_copy(ei_h.at[wid, 1], sd1, isem1)
    step(0, bufs[0], bufs[1], wait_scatter=False)
    step(1, bufs[1], bufs[0], wait_scatter=True)

    def pair(p, carry):
        j = 2 * p
        step(j, bufs[0], bufs[1], wait_scatter=True)
        step(j + 1, bufs[1], bufs[0], wait_scatter=True)
        return carry

    lax.fori_loop(1, (NCHUNK - 1) // 2, pair, 0)

    wait_scatters(bufs[1])
    wait_gathers(bufs[0])
    process(bufs[0])
    wait_scatters(bufs[0])
    pltpu.make_async_copy(ei_h.at[wid, NCHUNK - 1], sd1, isem1).wait()

    plsc.subcore_barrier()

    for t in range(RPT // CK):
        rb = nbase + t * CK

        @pl.when(rb < N)
        def _():
            pltpu.sync_copy(out_acc.at[pl.ds(rb, CK)], rows0)
            pltpu.sync_copy(rows0, out_h.at[cid, pl.ds(rb, CK)])
            pltpu.sync_copy(den_acc.at[pl.ds(rb, CK)], wrep0)
            pltpu.sync_copy(wrep0, den_h.at[cid, pl.ds(rb, CK)])


def _mid_body(p_ref, d_ref, b_ref, w_ref, att_ref, h_ref, ap_ref, mx_ref):
    i = pl.program_id(0)
    s = p_ref[0] + p_ref[1]
    den = d_ref[0, :, 0:1] + d_ref[1, :, 0:1]
    y = s / (den + 1e-16) + b_ref[...]
    y = jnp.where(y > 0.0, y, jnp.exp(jnp.minimum(y, 0.0)) - 1.0)
    h = jnp.dot(y, w_ref[...], preferred_element_type=f32)
    h_ref[...] = h
    ap = jnp.dot(h, att_ref[...], preferred_element_type=f32)
    ap_ref[...] = ap
    cur = jnp.max(ap, axis=0, keepdims=True)

    @pl.when(i == 0)
    def _():
        mx_ref[...] = cur

    @pl.when(i != 0)
    def _():
        mx_ref[...] = jnp.maximum(mx_ref[...], cur)


def _mid(p, d, b, W, att2):
    BN = 1000
    return pl.pallas_call(
        _mid_body,
        grid=(N // BN,),
        in_specs=[pl.BlockSpec((2, BN, D), lambda i: (0, i, 0)),
                  pl.BlockSpec((2, BN, 16), lambda i: (0, i, 0)),
                  pl.BlockSpec((1, D), lambda i: (0, 0)),
                  pl.BlockSpec((D, D), lambda i: (0, 0)),
                  pl.BlockSpec((D, 16), lambda i: (0, 0))],
        out_specs=[pl.BlockSpec((BN, D), lambda i: (i, 0)),
                   pl.BlockSpec((BN, 16), lambda i: (i, 0)),
                   pl.BlockSpec((1, 16), lambda i: (0, 0))],
        out_shape=[jax.ShapeDtypeStruct((N, D), f32),
                   jax.ShapeDtypeStruct((N, 16), f32),
                   jax.ShapeDtypeStruct((1, 16), f32)],
    )(p, d, b, W, att2)


def _combine_body(p_ref, d_ref, b_ref, o_ref, *, elu):
    s = p_ref[0] + p_ref[1]
    den = d_ref[0, :, 0:1] + d_ref[1, :, 0:1]
    y = s / (den + 1e-16) + b_ref[...]
    if elu:
        y = jnp.where(y > 0.0, y, jnp.exp(jnp.minimum(y, 0.0)) - 1.0)
    o_ref[...] = y


def _combine(p, d, b, elu):
    BN = 1000
    return pl.pallas_call(
        functools.partial(_combine_body, elu=elu),
        grid=(N // BN,),
        in_specs=[pl.BlockSpec((2, BN, D), lambda i: (0, i, 0)),
                  pl.BlockSpec((2, BN, 16), lambda i: (0, i, 0)),
                  pl.BlockSpec((1, D), lambda i: (0, 0))],
        out_specs=pl.BlockSpec((BN, D), lambda i: (i, 0)),
        out_shape=jax.ShapeDtypeStruct((N, D), f32),
    )(p, d, b)


def _shift(mx):
    c = mx[0, 0] + mx[0, 1]
    c = jnp.where(c >= 0.0, c, 0.2 * c)
    return jnp.broadcast_to(c, (16,))


def kernel(x, edge_index, W1, att_src1, att_dst1, b1, W2, att_src2, att_dst2, b2):
    src = edge_index[0].astype(i32).reshape(NW, NCHUNK, 1, CK)
    dst = edge_index[1].astype(i32).reshape(NW, NCHUNK, 1, CK)
    ei = jnp.concatenate([src, dst], axis=3)
    att21 = jnp.zeros((D, 16), f32).at[:, 0].set(att_src1).at[:, 1].set(att_dst1)
    att22 = jnp.zeros((D, 16), f32).at[:, 0].set(att_src2).at[:, 1].set(att_dst2)

    h1, ap1, mx1 = _front(x, W1, att21)
    outp1, denp1 = _sc_edge(ap1, _shift(mx1), ei, h1)
    h2, ap2, mx2 = _mid(outp1, denp1, b1.reshape(1, D), W2, att22)
    outp2, denp2 = _sc_edge(ap2, _shift(mx2), ei, h2)
    return _combine(outp2, denp2, b2.reshape(1, D), elu=False)
